# baseline (speedup 1.0000x reference)
import functools

import jax
import jax.numpy as jnp
from jax import lax
from jax.experimental import pallas as pl
from jax.experimental.pallas import tpu as pltpu
from jax.experimental.pallas import tpu_sc as plsc

VOCAB = 1000000
EMBED_DIM = 32
BATCH = 4096
SEQ_LEN = 200

NUM_CORES = 2
NUM_SUBCORES = 16
NW = NUM_CORES * NUM_SUBCORES
CHUNK = 128
NCH = SEQ_LEN
NBUF = 4
LOOKAHEAD = 2
GROUPS = CHUNK // 16

_MESH = plsc.VectorSubcoreMesh(
    core_axis_name="c", subcore_axis_name="s",
    num_cores=NUM_CORES, num_subcores=NUM_SUBCORES,
)


@functools.partial(
    pl.kernel,
    out_type=jax.ShapeDtypeStruct((SEQ_LEN, EMBED_DIM, BATCH), jnp.float32),
    mesh=_MESH,
    scratch_types=[
        pltpu.VMEM((SEQ_LEN, CHUNK), jnp.int32),
        pltpu.VMEM((NBUF, CHUNK), jnp.int32),
        pltpu.VMEM((NBUF, CHUNK, 128), jnp.float32),
        pltpu.VMEM((NBUF, EMBED_DIM, CHUNK), jnp.float32),
        pltpu.SemaphoreType.DMA((NBUF,)),
        pltpu.SemaphoreType.DMA((NBUF,)),
    ],
    compiler_params=pltpu.CompilerParams(
        use_tc_tiling_on_sc=True, needs_layout_passes=False),
)
def _sc_gather(data_hbm, tbl_hbm, out_hbm, idx_v, g_v, rows_v, trans_v,
               gsem, wsem):
    wid = lax.axis_index("s") * NUM_CORES + lax.axis_index("c")
    b0 = wid * CHUNK
    pltpu.sync_copy(data_hbm.at[:, pl.ds(b0, CHUNK)], idx_v)

    lane = lax.iota(jnp.int32, 16)

    def issue_gather(j, slot):
        for bg in range(GROUPS):
            v = idx_v[j, pl.ds(bg * 16, 16)]
            g_v[slot, pl.ds(bg * 16, 16)] = lax.shift_right_logical(v, 2)
        pltpu.async_copy(tbl_hbm.at[g_v.at[slot]], rows_v.at[slot],
                         gsem.at[slot])

    for b in range(LOOKAHEAD):
        issue_gather(b, b)

    @pl.loop(0, NCH)
    def _(j):
        b = lax.rem(j, NBUF)
        pltpu.make_async_copy(
            tbl_hbm.at[g_v.at[b]], rows_v.at[b], gsem.at[b]).wait()

        qcol = [
            (idx_v[j, pl.ds(bg * 16, 16)] & 3) * 32 for bg in range(GROUPS)
        ]
        bvec = [lane + bg * 16 for bg in range(GROUPS)]

        @pl.loop(0, EMBED_DIM, unroll=4)
        def _(f):
            for bg in range(GROUPS):
                x = plsc.load_gather(rows_v.at[b], [bvec[bg], qcol[bg] + f])
                trans_v[b, f, pl.ds(bg * 16, 16)] = x

        pltpu.async_copy(
            trans_v.at[b], out_hbm.at[j, :, pl.ds(b0, CHUNK)], wsem.at[b])

        jg = j + LOOKAHEAD
        bg_slot = lax.rem(jg, NBUF)

        @pl.when(jg < NCH)
        def _():
            @pl.when(jg >= NBUF)
            def _():
                jw = jg - NBUF
                pltpu.make_async_copy(
                    trans_v.at[bg_slot],
                    out_hbm.at[jw, :, pl.ds(b0, CHUNK)],
                    wsem.at[bg_slot]).wait()

            issue_gather(jg, bg_slot)

    for t in range(NBUF):
        jw = NCH - NBUF + t
        b = jw % NBUF
        pltpu.make_async_copy(
            trans_v.at[b], out_hbm.at[jw, :, pl.ds(b0, CHUNK)],
            wsem.at[b]).wait()


def kernel(data, table):
    data_sm = jnp.transpose(data).astype(jnp.int32)
    tbl_g = jnp.reshape(table, (VOCAB // 4, 128))
    out_sm = _sc_gather(data_sm, tbl_g)
    return jnp.transpose(out_sm, (2, 0, 1))
